# baseline (speedup 1.0000x reference)
import functools

import jax
import jax.numpy as jnp
from jax.experimental import pallas as pl
from jax.experimental.pallas import tpu as pltpu

_K0 = 0.11831801
_K1 = 0.14776132

_CROP = 5


def _loss_body(x_ref, y_ref, o_ref, *, w_blk, H, W):
    j = pl.program_id(1)
    x = x_ref[0, 0]
    y = y_ref[0, 0]

    xm = pltpu.roll(x, 1, 0)
    xp = pltpu.roll(x, H - 1, 0)
    ym = pltpu.roll(y, 1, 0)
    yp = pltpu.roll(y, H - 1, 0)

    mu1 = _K0 * (xm + xp) + _K1 * x
    mu2 = _K0 * (ym + yp) + _K1 * y
    c2x = _K0 * (xm * xm + xp * xp) + _K1 * (x * x)
    c2y = _K0 * (ym * ym + yp * yp) + _K1 * (y * y)
    cxy = _K0 * (xm * ym + xp * yp) + _K1 * (x * y)

    loss = ((c2x - mu1 * mu1) * (c2y - mu2 * mu2)
            - 2.0 * (cxy - mu1 * mu2))

    row = jax.lax.broadcasted_iota(jnp.int32, (H, w_blk), 0)
    col = j * w_blk + jax.lax.broadcasted_iota(jnp.int32, (H, w_blk), 1)
    mask = ((row >= _CROP) & (row < H - _CROP)
            & (col >= _CROP) & (col < W - _CROP))

    s = jnp.sum(jnp.where(mask, loss, 0.0)) * (1.0 / 128.0)
    o_ref[0, 0, :] = jnp.full((128,), s, dtype=jnp.float32)


def kernel(X, Y):
    B, C, H, W = X.shape
    w_blk = 512 if W % 512 == 0 else W
    nj = W // w_blk

    out = pl.pallas_call(
        functools.partial(_loss_body, w_blk=w_blk, H=H, W=W),
        out_shape=jax.ShapeDtypeStruct((B, 1, nj * 128), jnp.float32),
        grid=(B, nj),
        in_specs=[
            pl.BlockSpec((1, 1, H, w_blk), lambda b, j: (b, 0, 0, j)),
            pl.BlockSpec((1, 1, H, w_blk), lambda b, j: (b, 0, 0, j)),
        ],
        out_specs=pl.BlockSpec((1, 1, 128), lambda b, j: (b, 0, j)),
        compiler_params=pltpu.CompilerParams(
            dimension_semantics=("parallel", "parallel"),
        ),
        name="ssim_loss",
    )(X, Y)

    n = jnp.float32(H - 2 * _CROP) * jnp.float32(W - 2 * _CROP)
    return jnp.sum(out) / n


# speedup vs baseline: 1.1442x; 1.1442x over previous
import functools

import jax
import jax.numpy as jnp
from jax.experimental import pallas as pl
from jax.experimental.pallas import tpu as pltpu

_K0 = 0.11831801
_K1 = 0.14776132

_CROP = 5


def _tile_loss(xm, xc, xp, ym, yc, yp):
    mu1 = _K0 * (xm + xp) + _K1 * xc
    mu2 = _K0 * (ym + yp) + _K1 * yc
    c2x = _K0 * (xm * xm + xp * xp) + _K1 * (xc * xc)
    c2y = _K0 * (ym * ym + yp * yp) + _K1 * (yc * yc)
    cxy = _K0 * (xm * ym + xp * yp) + _K1 * (xc * yc)
    return ((c2x - mu1 * mu1) * (c2y - mu2 * mu2)
            - 2.0 * (cxy - mu1 * mu2))


def _loss_body(x_ref, y_ref, o_ref, *, w_blk, H, W):
    j = pl.program_id(1)

    def body(i, acc):
        w = x_ref[0, 0, pl.ds((i - 1) * 8, 24), :]
        v = y_ref[0, 0, pl.ds((i - 1) * 8, 24), :]
        return acc + _tile_loss(w[7:15], w[8:16], w[9:17],
                                v[7:15], v[8:16], v[9:17])

    acc = jnp.zeros((8, w_blk), jnp.float32)
    acc = jax.lax.fori_loop(1, H // 8 - 1, body, acc)

    hx = x_ref[0, 0, 0:16, :]
    hy = y_ref[0, 0, 0:16, :]
    head = _tile_loss(hx[4:7], hx[5:8], hx[6:9], hy[4:7], hy[5:8], hy[6:9])
    tx = x_ref[0, 0, H - 16:H, :]
    ty = y_ref[0, 0, H - 16:H, :]
    tail = _tile_loss(tx[7:10], tx[8:11], tx[9:12],
                      ty[7:10], ty[8:11], ty[9:12])

    col = j * w_blk + jax.lax.broadcasted_iota(jnp.int32, (1, w_blk), 1)
    cm = ((col >= _CROP) & (col < W - _CROP)).astype(jnp.float32)

    s = (jnp.sum(acc * cm) + jnp.sum(head * cm) + jnp.sum(tail * cm))
    o_ref[0, 0, :] = jnp.full((128,), s * (1.0 / 128.0), dtype=jnp.float32)


def kernel(X, Y):
    B, C, H, W = X.shape
    w_blk = 512 if W % 512 == 0 else W
    nj = W // w_blk

    out = pl.pallas_call(
        functools.partial(_loss_body, w_blk=w_blk, H=H, W=W),
        out_shape=jax.ShapeDtypeStruct((B, 1, nj * 128), jnp.float32),
        grid=(B, nj),
        in_specs=[
            pl.BlockSpec((1, 1, H, w_blk), lambda b, j: (b, 0, 0, j)),
            pl.BlockSpec((1, 1, H, w_blk), lambda b, j: (b, 0, 0, j)),
        ],
        out_specs=pl.BlockSpec((1, 1, 128), lambda b, j: (b, 0, j)),
        compiler_params=pltpu.CompilerParams(
            dimension_semantics=("parallel", "parallel"),
        ),
        name="ssim_loss",
    )(X, Y)

    n = jnp.float32(H - 2 * _CROP) * jnp.float32(W - 2 * _CROP)
    return jnp.sum(out) / n


# speedup vs baseline: 1.3995x; 1.2231x over previous
import functools

import jax
import jax.numpy as jnp
from jax.experimental import pallas as pl
from jax.experimental.pallas import tpu as pltpu

_K0 = 0.11831801
_K1 = 0.14776132

_CROP = 5


def _tile_loss(xm, xc, xp, ym, yc, yp):
    mu1 = _K0 * (xm + xp) + _K1 * xc
    mu2 = _K0 * (ym + yp) + _K1 * yc
    c2x = _K0 * (xm * xm + xp * xp) + _K1 * (xc * xc)
    c2y = _K0 * (ym * ym + yp * yp) + _K1 * (yc * yc)
    cxy = _K0 * (xm * ym + xp * yp) + _K1 * (xc * yc)
    return ((c2x - mu1 * mu1) * (c2y - mu2 * mu2)
            - 2.0 * (cxy - mu1 * mu2))


def _loss_body(x_ref, y_ref, o_ref, *, w_blk, H, W):
    j = pl.program_id(1)

    def body(i, acc):
        w = x_ref[0, 0, pl.ds((2 * i - 1) * 8, 32), :]
        v = y_ref[0, 0, pl.ds((2 * i - 1) * 8, 32), :]
        return acc + _tile_loss(w[7:23], w[8:24], w[9:25],
                                v[7:23], v[8:24], v[9:25])

    acc = jnp.zeros((16, w_blk), jnp.float32)
    acc = jax.lax.fori_loop(1, H // 16 - 1, body, acc)

    hx = x_ref[0, 0, 0:24, :]
    hy = y_ref[0, 0, 0:24, :]
    head = _tile_loss(hx[4:15], hx[5:16], hx[6:17],
                      hy[4:15], hy[5:16], hy[6:17])
    tx = x_ref[0, 0, H - 24:H, :]
    ty = y_ref[0, 0, H - 24:H, :]
    tail = _tile_loss(tx[7:18], tx[8:19], tx[9:20],
                      ty[7:18], ty[8:19], ty[9:20])

    col = j * w_blk + jax.lax.broadcasted_iota(jnp.int32, (1, w_blk), 1)
    cm = ((col >= _CROP) & (col < W - _CROP)).astype(jnp.float32)

    s = (jnp.sum(acc * cm) + jnp.sum(head * cm) + jnp.sum(tail * cm))
    o_ref[0, 0, :] = jnp.full((128,), s * (1.0 / 128.0), dtype=jnp.float32)


def kernel(X, Y):
    B, C, H, W = X.shape
    w_blk = 512 if W % 512 == 0 else W
    nj = W // w_blk

    out = pl.pallas_call(
        functools.partial(_loss_body, w_blk=w_blk, H=H, W=W),
        out_shape=jax.ShapeDtypeStruct((B, 1, nj * 128), jnp.float32),
        grid=(B, nj),
        in_specs=[
            pl.BlockSpec((1, 1, H, w_blk), lambda b, j: (b, 0, 0, j)),
            pl.BlockSpec((1, 1, H, w_blk), lambda b, j: (b, 0, 0, j)),
        ],
        out_specs=pl.BlockSpec((1, 1, 128), lambda b, j: (b, 0, j)),
        compiler_params=pltpu.CompilerParams(
            dimension_semantics=("parallel", "parallel"),
        ),
        name="ssim_loss",
    )(X, Y)

    n = jnp.float32(H - 2 * _CROP) * jnp.float32(W - 2 * _CROP)
    return jnp.sum(out) / n


# speedup vs baseline: 1.5458x; 1.1046x over previous
import functools

import jax
import jax.numpy as jnp
from jax.experimental import pallas as pl
from jax.experimental.pallas import tpu as pltpu

_K0 = 0.11831801
_K1 = 0.14776132

_CROP = 5


def _tile_loss(xm, xc, xp, ym, yc, yp):
    mu1 = _K0 * (xm + xp) + _K1 * xc
    mu2 = _K0 * (ym + yp) + _K1 * yc
    c2x = _K0 * (xm * xm + xp * xp) + _K1 * (xc * xc)
    c2y = _K0 * (ym * ym + yp * yp) + _K1 * (yc * yc)
    cxy = _K0 * (xm * ym + xp * yp) + _K1 * (xc * yc)
    return ((c2x - mu1 * mu1) * (c2y - mu2 * mu2)
            - 2.0 * (cxy - mu1 * mu2))


def _loss_body(x_ref, y_ref, o_ref, *, w_blk, H, W):
    j = pl.program_id(1)

    def body(i, acc):
        w = x_ref[0, 0, pl.ds((4 * i - 1) * 8, 48), :]
        v = y_ref[0, 0, pl.ds((4 * i - 1) * 8, 48), :]
        return acc + _tile_loss(w[7:39], w[8:40], w[9:41],
                                v[7:39], v[8:40], v[9:41])

    acc = jnp.zeros((32, w_blk), jnp.float32)
    acc = jax.lax.fori_loop(1, H // 32 - 1, body, acc)

    hx = x_ref[0, 0, 0:40, :]
    hy = y_ref[0, 0, 0:40, :]
    head = _tile_loss(hx[4:31], hx[5:32], hx[6:33],
                      hy[4:31], hy[5:32], hy[6:33])
    tx = x_ref[0, 0, H - 40:H, :]
    ty = y_ref[0, 0, H - 40:H, :]
    tail = _tile_loss(tx[7:34], tx[8:35], tx[9:36],
                      ty[7:34], ty[8:35], ty[9:36])

    col = j * w_blk + jax.lax.broadcasted_iota(jnp.int32, (1, w_blk), 1)
    cm = ((col >= _CROP) & (col < W - _CROP)).astype(jnp.float32)

    s = (jnp.sum(acc * cm) + jnp.sum(head * cm) + jnp.sum(tail * cm))
    o_ref[0, 0, :] = jnp.full((128,), s * (1.0 / 128.0), dtype=jnp.float32)


def kernel(X, Y):
    B, C, H, W = X.shape
    w_blk = 512 if W % 512 == 0 else W
    nj = W // w_blk

    out = pl.pallas_call(
        functools.partial(_loss_body, w_blk=w_blk, H=H, W=W),
        out_shape=jax.ShapeDtypeStruct((B, 1, nj * 128), jnp.float32),
        grid=(B, nj),
        in_specs=[
            pl.BlockSpec((1, 1, H, w_blk), lambda b, j: (b, 0, 0, j)),
            pl.BlockSpec((1, 1, H, w_blk), lambda b, j: (b, 0, 0, j)),
        ],
        out_specs=pl.BlockSpec((1, 1, 128), lambda b, j: (b, 0, j)),
        compiler_params=pltpu.CompilerParams(
            dimension_semantics=("parallel", "parallel"),
        ),
        name="ssim_loss",
    )(X, Y)

    n = jnp.float32(H - 2 * _CROP) * jnp.float32(W - 2 * _CROP)
    return jnp.sum(out) / n
